# baseline (speedup 1.0000x reference)
import functools

import jax
import jax.numpy as jnp
from jax import lax
from jax.experimental import pallas as pl
from jax.experimental.pallas import tpu as pltpu
from jax.experimental.pallas import tpu_sc as plsc

_N = 16384
_D = 32
_V = 1_000_000
_NC = 2
_NS = 16
_NW = _NC * _NS
_BPW = _N // _NW
_CHUNK = 128
_NCHUNK = _BPW // _CHUNK
_GRP = 16
_R = 128


def _sc_scores(u2, i2, j2, pidx, iidx, jidx, p_lin, q_lin, bu, bi):
    mesh = plsc.VectorSubcoreMesh(core_axis_name="c", subcore_axis_name="s")

    @functools.partial(
        pl.kernel,
        mesh=mesh,
        compiler_params=pltpu.CompilerParams(
            use_tc_tiling_on_sc=False, needs_layout_passes=False
        ),
        out_type=(
            jax.ShapeDtypeStruct((_R, _R), jnp.float32),
            jax.ShapeDtypeStruct((_R, _R), jnp.float32),
        ),
        scratch_types=[
            pltpu.VMEM((_NCHUNK, _CHUNK), jnp.int32),
            pltpu.VMEM((_NCHUNK, _CHUNK), jnp.int32),
            pltpu.VMEM((_NCHUNK, _CHUNK), jnp.int32),
            pltpu.VMEM((_NCHUNK, _D * _CHUNK), jnp.int32),
            pltpu.VMEM((_NCHUNK, _D * _CHUNK), jnp.int32),
            pltpu.VMEM((_NCHUNK, _D * _CHUNK), jnp.int32),
            pltpu.VMEM((2, _D * _CHUNK), jnp.float32),
            pltpu.VMEM((2, _D * _CHUNK), jnp.float32),
            pltpu.VMEM((2, _D * _CHUNK), jnp.float32),
            pltpu.VMEM((2, _CHUNK), jnp.float32),
            pltpu.VMEM((2, _CHUNK), jnp.float32),
            pltpu.VMEM((2, _CHUNK), jnp.float32),
            pltpu.VMEM((_NCHUNK, _CHUNK), jnp.float32),
            pltpu.VMEM((_NCHUNK, _CHUNK), jnp.float32),
            pltpu.SemaphoreType.DMA,
        ],
    )
    def k(u_hbm, i_hbm, j_hbm, pidx_hbm, iidx_hbm, jidx_hbm,
          p_hbm, q_hbm, bu_hbm, bi_hbm,
          rui_hbm, ruj_hbm,
          u_v, i_v, j_v, pix_v, iix_v, jix_v, pu_v, qi_v, qj_v,
          bu_v, bii_v, bij_v, rui_v, ruj_v, sem):
        wid = lax.axis_index("s") * _NC + lax.axis_index("c")
        row0 = wid * _NCHUNK

        pltpu.sync_copy(u_hbm.at[pl.ds(row0, _NCHUNK)], u_v)
        pltpu.sync_copy(i_hbm.at[pl.ds(row0, _NCHUNK)], i_v)
        pltpu.sync_copy(j_hbm.at[pl.ds(row0, _NCHUNK)], j_v)
        pltpu.sync_copy(pidx_hbm.at[pl.ds(row0, _NCHUNK)], pix_v)
        pltpu.sync_copy(iidx_hbm.at[pl.ds(row0, _NCHUNK)], iix_v)
        pltpu.sync_copy(jidx_hbm.at[pl.ds(row0, _NCHUNK)], jix_v)

        def fire(c):
            b = c % 2
            return (
                pltpu.async_copy(p_hbm.at[pix_v.at[c]], pu_v.at[b], sem),
                pltpu.async_copy(q_hbm.at[iix_v.at[c]], qi_v.at[b], sem),
                pltpu.async_copy(q_hbm.at[jix_v.at[c]], qj_v.at[b], sem),
                pltpu.async_copy(bu_hbm.at[u_v.at[c]], bu_v.at[b], sem),
                pltpu.async_copy(bi_hbm.at[i_v.at[c]], bii_v.at[b], sem),
                pltpu.async_copy(bi_hbm.at[j_v.at[c]], bij_v.at[b], sem),
            )

        lane = lax.broadcasted_iota(jnp.int32, (_GRP,), 0)
        cps = {0: fire(0)}
        for c in range(_NCHUNK):
            if c + 1 < _NCHUNK:
                cps[c + 1] = fire(c + 1)
            for cp in cps.pop(c):
                cp.wait()
            b = c % 2
            bvec = jnp.full((_GRP,), b, jnp.int32)
            cvec = jnp.full((_GRP,), c, jnp.int32)

            def body(g, carry, bvec=bvec, cvec=cvec):
                row = g * _GRP + lane
                acc_i = jnp.zeros((_GRP,), jnp.float32)
                acc_j = jnp.zeros((_GRP,), jnp.float32)
                for dd in range(_D):
                    pos = row + dd * _CHUNK
                    pu_d = plsc.load_gather(pu_v, [bvec, pos])
                    qi_d = plsc.load_gather(qi_v, [bvec, pos])
                    qj_d = plsc.load_gather(qj_v, [bvec, pos])
                    acc_i = acc_i + pu_d * qi_d
                    acc_j = acc_j + pu_d * qj_d
                b_u = plsc.load_gather(bu_v, [bvec, row])
                b_i = plsc.load_gather(bii_v, [bvec, row])
                b_j = plsc.load_gather(bij_v, [bvec, row])
                plsc.store_scatter(rui_v, [cvec, row], b_u + b_i + acc_i)
                plsc.store_scatter(ruj_v, [cvec, row], b_u + b_j + acc_j)
                return carry

            lax.fori_loop(0, _CHUNK // _GRP, body, 0)

        pltpu.sync_copy(rui_v, rui_hbm.at[pl.ds(row0, _NCHUNK)])
        pltpu.sync_copy(ruj_v, ruj_hbm.at[pl.ds(row0, _NCHUNK)])

    return k(u2, i2, j2, pidx, iidx, jidx, p_lin, q_lin, bu, bi)


def _loss_body(m_ref, rui_ref, ruj_ref, ui_ref, uj_ref, out_ref):
    m = m_ref[0]
    rui = rui_ref[...] + m
    ruj = ruj_ref[...] + m
    r = rui - ruj
    bpr = jnp.maximum(-r, 0.0) + jnp.log1p(jnp.exp(-jnp.abs(r)))
    d1 = rui - ui_ref[...]
    a1 = jnp.abs(d1)
    s1 = jnp.where(a1 < 1.0, 0.5 * d1 * d1, a1 - 0.5)
    d2 = ruj - uj_ref[...]
    a2 = jnp.abs(d2)
    s2 = jnp.where(a2 < 1.0, 0.5 * d2 * d2, a2 - 0.5)
    out_ref[0, 0] = (0.7 * jnp.mean(bpr)
                     + 0.3 * 0.5 * (jnp.mean(s1) + jnp.mean(s2)))


def kernel(u, i, j, ui, uj, m, bu, bi, p, q):
    u32 = u.astype(jnp.int32)
    i32 = i.astype(jnp.int32)
    j32 = j.astype(jnp.int32)
    u2 = jnp.reshape(u32, (_R, _R))
    i2 = jnp.reshape(i32, (_R, _R))
    j2 = jnp.reshape(j32, (_R, _R))
    p_lin = jnp.reshape(p.T, (_V * _D,))
    q_lin = jnp.reshape(q.T, (_V * _D,))
    doff = (jnp.arange(_D, dtype=jnp.int32) * _V)[None, :, None]
    pidx = jnp.reshape(jnp.reshape(u32, (_R, 1, _R)) + doff, (_R, _D * _R))
    iidx = jnp.reshape(jnp.reshape(i32, (_R, 1, _R)) + doff, (_R, _D * _R))
    jidx = jnp.reshape(jnp.reshape(j32, (_R, 1, _R)) + doff, (_R, _D * _R))
    rui, ruj = _sc_scores(u2, i2, j2, pidx, iidx, jidx, p_lin, q_lin, bu, bi)
    out = pl.pallas_call(
        _loss_body,
        out_shape=jax.ShapeDtypeStruct((1, 1), jnp.float32),
        in_specs=[
            pl.BlockSpec(memory_space=pltpu.SMEM),
            pl.BlockSpec(memory_space=pltpu.VMEM),
            pl.BlockSpec(memory_space=pltpu.VMEM),
            pl.BlockSpec(memory_space=pltpu.VMEM),
            pl.BlockSpec(memory_space=pltpu.VMEM),
        ],
        out_specs=pl.BlockSpec(memory_space=pltpu.SMEM),
    )(m, rui, ruj, jnp.reshape(ui, (_R, _R)), jnp.reshape(uj, (_R, _R)))
    return out[0, 0]


# speedup vs baseline: 4.2608x; 4.2608x over previous
import functools

import jax
import jax.numpy as jnp
from jax import lax
from jax.experimental import pallas as pl
from jax.experimental.pallas import tpu as pltpu
from jax.experimental.pallas import tpu_sc as plsc

_N = 16384
_D = 32
_NC = 2
_NS = 16
_NW = _NC * _NS
_BPW = _N // _NW
_CHUNK = 128
_NCHUNK = _BPW // _CHUNK
_GRP = 16
_R = 128


def _sc_scores(u2, i2, j2, w, bu, bi):
    mesh = plsc.VectorSubcoreMesh(core_axis_name="c", subcore_axis_name="s")

    @functools.partial(
        pl.kernel,
        mesh=mesh,
        compiler_params=pltpu.CompilerParams(
            use_tc_tiling_on_sc=False, needs_layout_passes=False
        ),
        out_type=(
            jax.ShapeDtypeStruct((_R, _R), jnp.float32),
            jax.ShapeDtypeStruct((_R, _R), jnp.float32),
        ),
        scratch_types=[
            pltpu.VMEM((_NCHUNK, _CHUNK), jnp.int32),
            pltpu.VMEM((_NCHUNK, _CHUNK), jnp.int32),
            pltpu.VMEM((_NCHUNK, _CHUNK), jnp.int32),
            pltpu.VMEM((2, _CHUNK, 2 * _D), jnp.float32),
            pltpu.VMEM((2, _CHUNK, 2 * _D), jnp.float32),
            pltpu.VMEM((2, _CHUNK, 2 * _D), jnp.float32),
            pltpu.VMEM((2, _CHUNK), jnp.float32),
            pltpu.VMEM((2, _CHUNK), jnp.float32),
            pltpu.VMEM((2, _CHUNK), jnp.float32),
            pltpu.VMEM((_NCHUNK, _CHUNK), jnp.float32),
            pltpu.VMEM((_NCHUNK, _CHUNK), jnp.float32),
            pltpu.SemaphoreType.DMA,
        ],
    )
    def k(u_hbm, i_hbm, j_hbm, w_hbm, bu_hbm, bi_hbm,
          rui_hbm, ruj_hbm,
          u_v, i_v, j_v, pu_v, qi_v, qj_v, bu_v, bii_v, bij_v,
          rui_v, ruj_v, sem):
        wid = lax.axis_index("s") * _NC + lax.axis_index("c")
        row0 = wid * _NCHUNK

        pltpu.sync_copy(u_hbm.at[pl.ds(row0, _NCHUNK)], u_v)
        pltpu.sync_copy(i_hbm.at[pl.ds(row0, _NCHUNK)], i_v)
        pltpu.sync_copy(j_hbm.at[pl.ds(row0, _NCHUNK)], j_v)

        def fire(c):
            b = c % 2
            return (
                pltpu.async_copy(w_hbm.at[u_v.at[c]], pu_v.at[b], sem),
                pltpu.async_copy(w_hbm.at[i_v.at[c]], qi_v.at[b], sem),
                pltpu.async_copy(w_hbm.at[j_v.at[c]], qj_v.at[b], sem),
                pltpu.async_copy(bu_hbm.at[u_v.at[c]], bu_v.at[b], sem),
                pltpu.async_copy(bi_hbm.at[i_v.at[c]], bii_v.at[b], sem),
                pltpu.async_copy(bi_hbm.at[j_v.at[c]], bij_v.at[b], sem),
            )

        lane = lax.broadcasted_iota(jnp.int32, (_GRP,), 0)
        cps = {0: fire(0)}
        for c in range(_NCHUNK):
            if c + 1 < _NCHUNK:
                cps[c + 1] = fire(c + 1)
            for cp in cps.pop(c):
                cp.wait()
            b = c % 2
            bvec = jnp.full((_GRP,), b, jnp.int32)
            cvec = jnp.full((_GRP,), c, jnp.int32)

            def body(g, carry, bvec=bvec, cvec=cvec):
                row = g * _GRP + lane
                acc_i = jnp.zeros((_GRP,), jnp.float32)
                acc_j = jnp.zeros((_GRP,), jnp.float32)
                for dd in range(_D):
                    pcol = jnp.full((_GRP,), dd, jnp.int32)
                    qcol = jnp.full((_GRP,), _D + dd, jnp.int32)
                    pu_d = plsc.load_gather(pu_v, [bvec, row, pcol])
                    qi_d = plsc.load_gather(qi_v, [bvec, row, qcol])
                    qj_d = plsc.load_gather(qj_v, [bvec, row, qcol])
                    acc_i = acc_i + pu_d * qi_d
                    acc_j = acc_j + pu_d * qj_d
                b_u = plsc.load_gather(bu_v, [bvec, row])
                b_i = plsc.load_gather(bii_v, [bvec, row])
                b_j = plsc.load_gather(bij_v, [bvec, row])
                plsc.store_scatter(rui_v, [cvec, row], b_u + b_i + acc_i)
                plsc.store_scatter(ruj_v, [cvec, row], b_u + b_j + acc_j)
                return carry

            lax.fori_loop(0, _CHUNK // _GRP, body, 0)

        pltpu.sync_copy(rui_v, rui_hbm.at[pl.ds(row0, _NCHUNK)])
        pltpu.sync_copy(ruj_v, ruj_hbm.at[pl.ds(row0, _NCHUNK)])

    return k(u2, i2, j2, w, bu, bi)


def _loss_body(m_ref, rui_ref, ruj_ref, ui_ref, uj_ref, out_ref):
    m = m_ref[0]
    rui = rui_ref[...] + m
    ruj = ruj_ref[...] + m
    r = rui - ruj
    bpr = jnp.maximum(-r, 0.0) + jnp.log1p(jnp.exp(-jnp.abs(r)))
    d1 = rui - ui_ref[...]
    a1 = jnp.abs(d1)
    s1 = jnp.where(a1 < 1.0, 0.5 * d1 * d1, a1 - 0.5)
    d2 = ruj - uj_ref[...]
    a2 = jnp.abs(d2)
    s2 = jnp.where(a2 < 1.0, 0.5 * d2 * d2, a2 - 0.5)
    out_ref[0, 0] = (0.7 * jnp.mean(bpr)
                     + 0.3 * 0.5 * (jnp.mean(s1) + jnp.mean(s2)))


def kernel(u, i, j, ui, uj, m, bu, bi, p, q):
    u2 = jnp.reshape(u.astype(jnp.int32), (_R, _R))
    i2 = jnp.reshape(i.astype(jnp.int32), (_R, _R))
    j2 = jnp.reshape(j.astype(jnp.int32), (_R, _R))
    w = jnp.concatenate([p, q], axis=1)
    rui, ruj = _sc_scores(u2, i2, j2, w, bu, bi)
    out = pl.pallas_call(
        _loss_body,
        out_shape=jax.ShapeDtypeStruct((1, 1), jnp.float32),
        in_specs=[
            pl.BlockSpec(memory_space=pltpu.SMEM),
            pl.BlockSpec(memory_space=pltpu.VMEM),
            pl.BlockSpec(memory_space=pltpu.VMEM),
            pl.BlockSpec(memory_space=pltpu.VMEM),
            pl.BlockSpec(memory_space=pltpu.VMEM),
        ],
        out_specs=pl.BlockSpec(memory_space=pltpu.SMEM),
    )(m, rui, ruj, jnp.reshape(ui, (_R, _R)), jnp.reshape(uj, (_R, _R)))
    return out[0, 0]


# speedup vs baseline: 5.6417x; 1.3241x over previous
import functools

import jax
import jax.numpy as jnp
from jax import lax
from jax.experimental import pallas as pl
from jax.experimental.pallas import tpu as pltpu
from jax.experimental.pallas import tpu_sc as plsc

_N = 16384
_D = 32
_NC = 2
_NS = 16
_NW = _NC * _NS
_BPW = _N // _NW
_CHUNK = 128
_NCHUNK = _BPW // _CHUNK
_GRP = 16
_R = 128


def _sc_scores(u2, i2, j2, p, q, bu, bi):
    mesh = plsc.VectorSubcoreMesh(core_axis_name="c", subcore_axis_name="s")

    @functools.partial(
        pl.kernel,
        mesh=mesh,
        compiler_params=pltpu.CompilerParams(
            use_tc_tiling_on_sc=False, needs_layout_passes=False
        ),
        out_type=(
            jax.ShapeDtypeStruct((_R, _R), jnp.float32),
            jax.ShapeDtypeStruct((_R, _R), jnp.float32),
        ),
        scratch_types=[
            pltpu.VMEM((_NCHUNK, _CHUNK), jnp.int32),
            pltpu.VMEM((_NCHUNK, _CHUNK), jnp.int32),
            pltpu.VMEM((_NCHUNK, _CHUNK), jnp.int32),
            pltpu.VMEM((2, _CHUNK, _D), jnp.float32),
            pltpu.VMEM((2, _CHUNK, _D), jnp.float32),
            pltpu.VMEM((2, _CHUNK, _D), jnp.float32),
            pltpu.VMEM((2, _CHUNK), jnp.float32),
            pltpu.VMEM((2, _CHUNK), jnp.float32),
            pltpu.VMEM((2, _CHUNK), jnp.float32),
            pltpu.VMEM((_NCHUNK, _CHUNK), jnp.float32),
            pltpu.VMEM((_NCHUNK, _CHUNK), jnp.float32),
            pltpu.SemaphoreType.DMA,
        ],
    )
    def k(u_hbm, i_hbm, j_hbm, p_hbm, q_hbm, bu_hbm, bi_hbm,
          rui_hbm, ruj_hbm,
          u_v, i_v, j_v, pu_v, qi_v, qj_v, bu_v, bii_v, bij_v,
          rui_v, ruj_v, sem):
        wid = lax.axis_index("s") * _NC + lax.axis_index("c")
        row0 = wid * _NCHUNK

        pltpu.sync_copy(u_hbm.at[pl.ds(row0, _NCHUNK)], u_v)
        pltpu.sync_copy(i_hbm.at[pl.ds(row0, _NCHUNK)], i_v)
        pltpu.sync_copy(j_hbm.at[pl.ds(row0, _NCHUNK)], j_v)

        def fire(c):
            b = c % 2
            return (
                pltpu.async_copy(p_hbm.at[u_v.at[c]], pu_v.at[b], sem),
                pltpu.async_copy(q_hbm.at[i_v.at[c]], qi_v.at[b], sem),
                pltpu.async_copy(q_hbm.at[j_v.at[c]], qj_v.at[b], sem),
                pltpu.async_copy(bu_hbm.at[u_v.at[c]], bu_v.at[b], sem),
                pltpu.async_copy(bi_hbm.at[i_v.at[c]], bii_v.at[b], sem),
                pltpu.async_copy(bi_hbm.at[j_v.at[c]], bij_v.at[b], sem),
            )

        lane = lax.broadcasted_iota(jnp.int32, (_GRP,), 0)
        cps = {0: fire(0)}
        for c in range(_NCHUNK):
            if c + 1 < _NCHUNK:
                cps[c + 1] = fire(c + 1)
            for cp in cps.pop(c):
                cp.wait()
            b = c % 2
            bvec = jnp.full((_GRP,), b, jnp.int32)
            cvec = jnp.full((_GRP,), c, jnp.int32)

            def body(g, carry, bvec=bvec, cvec=cvec):
                row = g * _GRP + lane
                acc_i = jnp.zeros((_GRP,), jnp.float32)
                acc_j = jnp.zeros((_GRP,), jnp.float32)
                for dd in range(_D):
                    dcol = jnp.full((_GRP,), dd, jnp.int32)
                    pu_d = plsc.load_gather(pu_v, [bvec, row, dcol])
                    qi_d = plsc.load_gather(qi_v, [bvec, row, dcol])
                    qj_d = plsc.load_gather(qj_v, [bvec, row, dcol])
                    acc_i = acc_i + pu_d * qi_d
                    acc_j = acc_j + pu_d * qj_d
                b_u = plsc.load_gather(bu_v, [bvec, row])
                b_i = plsc.load_gather(bii_v, [bvec, row])
                b_j = plsc.load_gather(bij_v, [bvec, row])
                plsc.store_scatter(rui_v, [cvec, row], b_u + b_i + acc_i)
                plsc.store_scatter(ruj_v, [cvec, row], b_u + b_j + acc_j)
                return carry

            lax.fori_loop(0, _CHUNK // _GRP, body, 0)

        pltpu.sync_copy(rui_v, rui_hbm.at[pl.ds(row0, _NCHUNK)])
        pltpu.sync_copy(ruj_v, ruj_hbm.at[pl.ds(row0, _NCHUNK)])

    return k(u2, i2, j2, p, q, bu, bi)


def _loss_body(m_ref, rui_ref, ruj_ref, ui_ref, uj_ref, out_ref):
    m = m_ref[0]
    rui = rui_ref[...] + m
    ruj = ruj_ref[...] + m
    r = rui - ruj
    bpr = jnp.maximum(-r, 0.0) + jnp.log1p(jnp.exp(-jnp.abs(r)))
    d1 = rui - ui_ref[...]
    a1 = jnp.abs(d1)
    s1 = jnp.where(a1 < 1.0, 0.5 * d1 * d1, a1 - 0.5)
    d2 = ruj - uj_ref[...]
    a2 = jnp.abs(d2)
    s2 = jnp.where(a2 < 1.0, 0.5 * d2 * d2, a2 - 0.5)
    out_ref[0, 0] = (0.7 * jnp.mean(bpr)
                     + 0.3 * 0.5 * (jnp.mean(s1) + jnp.mean(s2)))


def kernel(u, i, j, ui, uj, m, bu, bi, p, q):
    u2 = jnp.reshape(u.astype(jnp.int32), (_R, _R))
    i2 = jnp.reshape(i.astype(jnp.int32), (_R, _R))
    j2 = jnp.reshape(j.astype(jnp.int32), (_R, _R))
    rui, ruj = _sc_scores(u2, i2, j2, p, q, bu, bi)
    out = pl.pallas_call(
        _loss_body,
        out_shape=jax.ShapeDtypeStruct((1, 1), jnp.float32),
        in_specs=[
            pl.BlockSpec(memory_space=pltpu.SMEM),
            pl.BlockSpec(memory_space=pltpu.VMEM),
            pl.BlockSpec(memory_space=pltpu.VMEM),
            pl.BlockSpec(memory_space=pltpu.VMEM),
            pl.BlockSpec(memory_space=pltpu.VMEM),
        ],
        out_specs=pl.BlockSpec(memory_space=pltpu.SMEM),
    )(m, rui, ruj, jnp.reshape(ui, (_R, _R)), jnp.reshape(uj, (_R, _R)))
    return out[0, 0]
